# Initial kernel scaffold; baseline (speedup 1.0000x reference)
import functools

import jax
import jax.numpy as jnp
from jax import lax
from jax.experimental import pallas as pl
from jax.experimental.pallas import tpu as pltpu
from jax.experimental.pallas import tpu_sc as plsc

B = 262144
C = 16
NC, NS, L = 2, 16, 16
NW = NC * NS
PPW = B // NW
S = 128
NCHUNK = PPW // S
NG = S // L
NSLOT = 24


def _axis_setup(c, scale_m1):
    k = 0.5 * scale_m1
    p = c * k + k
    p = jnp.minimum(jnp.maximum(p, 0.0), scale_m1)
    i0 = p.astype(jnp.int32)
    w = p - i0.astype(jnp.float32)
    i1 = jnp.minimum(i0 + 1, jnp.int32(scale_m1))
    return i0, i1, w


def _line_setup(c, n_m1):
    p = c * n_m1
    p = jnp.minimum(jnp.maximum(p, 0.0), n_m1)
    i0 = p.astype(jnp.int32)
    w = p - i0.astype(jnp.float32)
    i1 = jnp.minimum(i0 + 1, jnp.int32(n_m1))
    return i0, i1, w


def _body(xT, gridT, p0T, p1T, p2T, l0T, l1T, out,
          cbuf, idxbuf, wbuf, gbuf, obuf, sem):
    cid = lax.axis_index("c")
    sid = lax.axis_index("s")
    wid = sid * NC + cid
    base_w = wid * PPW
    tables = [gridT] * 8 + [p0T] * 4 + [p1T] * 4 + [p2T] * 4 + [l0T] * 2 + [l1T] * 2
    iota = lax.iota(jnp.int32, L)

    def chunk_body(t, carry):
        base = base_w + t * S
        for d in range(5):
            pltpu.sync_copy(xT.at[d, pl.ds(base, S)], cbuf.at[d])

        for g in range(NG):
            sl = pl.ds(g * L, L)
            c0 = cbuf[0, sl]
            c1 = cbuf[1, sl]
            c2 = cbuf[2, sl]
            c3 = cbuf[3, sl]
            c4 = cbuf[4, sl]

            slot = 0
            x0, x1, wx = _axis_setup(c0, 127.0)
            y0, y1, wy = _axis_setup(c1, 127.0)
            z0, z1, wz = _axis_setup(c2, 127.0)
            xi = (x0, x1)
            wxp = (1.0 - wx, wx)
            yi = (y0, y1)
            wyp = (1.0 - wy, wy)
            zi = (z0, z1)
            wzp = (1.0 - wz, wz)
            for a in range(2):
                za = zi[a] * 128
                for b in range(2):
                    rowbase = (za + yi[b]) * 128
                    wzy = wzp[a] * wyp[b]
                    for e in range(2):
                        idxbuf[slot, sl] = rowbase + xi[e]
                        wbuf[slot, sl] = wzy * wxp[e]
                        slot += 1

            axes255 = [_axis_setup(c0, 255.0), _axis_setup(c1, 255.0),
                       _axis_setup(c2, 255.0)]
            for (pa, pb) in ((1, 2), (0, 2), (0, 1)):
                u0, u1, wu = axes255[pa]
                v0, v1, wv_ = axes255[pb]
                ui = (u0, u1)
                wup = (1.0 - wu, wu)
                vi = (v0, v1)
                wvp = (1.0 - wv_, wv_)
                for b in range(2):
                    rowbase = vi[b] * 256
                    for e in range(2):
                        idxbuf[slot, sl] = rowbase + ui[e]
                        wbuf[slot, sl] = wvp[b] * wup[e]
                        slot += 1

            for cl in (c3, c4):
                i0, i1, wl = _line_setup(cl, 63.0)
                idxbuf[slot, sl] = i0
                wbuf[slot, sl] = 1.0 - wl
                slot += 1
                idxbuf[slot, sl] = i1
                wbuf[slot, sl] = wl
                slot += 1

        descs = [pltpu.async_copy(tables[k].at[idxbuf.at[k]], gbuf.at[k], sem)
                 for k in range(NSLOT)]
        for dsc in descs:
            dsc.wait()

        def group_body(g, carry2):
            rows = g * L + iota
            wv = [wbuf[k, pl.ds(g * L, L)] for k in range(NSLOT)]
            for ch in range(C):
                colc = jnp.full((L,), ch, jnp.int32)

                def gk(k):
                    return plsc.load_gather(gbuf.at[k], [rows, colc])

                acc = wv[0] * gk(0)
                for k in range(1, 8):
                    acc = acc + wv[k] * gk(k)
                for p in range(3):
                    b0 = 8 + 4 * p
                    pacc = wv[b0] * gk(b0)
                    for j in range(1, 4):
                        pacc = pacc + wv[b0 + j] * gk(b0 + j)
                    acc = acc * pacc
                l0v = wv[20] * gk(20) + wv[21] * gk(21)
                l1v = wv[22] * gk(22) + wv[23] * gk(23)
                par = l0v * l1v
                plsc.store_scatter(obuf, [rows, colc], acc)
                plsc.store_scatter(obuf, [rows, jnp.full((L,), C + ch, jnp.int32)], par)
            return carry2

        lax.fori_loop(0, NG, group_body, 0)
        pltpu.sync_copy(obuf, out.at[pl.ds(base, S)])
        return carry

    lax.fori_loop(0, NCHUNK, chunk_body, 0)


_sc_call = functools.partial(
    pl.kernel,
    out_type=jax.ShapeDtypeStruct((B, 2 * C), jnp.float32),
    mesh=plsc.VectorSubcoreMesh(core_axis_name="c", subcore_axis_name="s",
                                num_cores=NC, num_subcores=NS),
    scratch_types=[
        pltpu.VMEM((5, S), jnp.float32),
        pltpu.VMEM((NSLOT, S), jnp.int32),
        pltpu.VMEM((NSLOT, S), jnp.float32),
        pltpu.VMEM((NSLOT, S, C), jnp.float32),
        pltpu.VMEM((S, 2 * C), jnp.float32),
        pltpu.SemaphoreType.DMA,
    ],
)(_body)


def kernel(x, grid3d, plane0, plane1, plane2, line0, line1):
    xT = x.T
    gridT = jnp.transpose(grid3d, (1, 2, 3, 0)).reshape(-1, C)
    p0T = jnp.transpose(plane0, (1, 2, 0)).reshape(-1, C)
    p1T = jnp.transpose(plane1, (1, 2, 0)).reshape(-1, C)
    p2T = jnp.transpose(plane2, (1, 2, 0)).reshape(-1, C)
    l0T = line0.T
    l1T = line1.T
    return _sc_call(xT, gridT, p0T, p1T, p2T, l0T, l1T)


# speedup vs baseline: 23.2078x; 23.2078x over previous
import functools

import jax
import jax.numpy as jnp
from jax import lax
from jax.experimental import pallas as pl
from jax.experimental.pallas import tpu as pltpu
from jax.experimental.pallas import tpu_sc as plsc

B = 262144
C = 16
NC, NS, L = 2, 16, 16
NW = NC * NS
PPW = B // NW
S = 128
NCHUNK = PPW // S
NG = S // L
NSLOT = 24
OC = 2 * C


def _axis_setup(c, scale_m1):
    k = 0.5 * scale_m1
    p = c * k + k
    p = jnp.minimum(jnp.maximum(p, 0.0), scale_m1)
    i0 = p.astype(jnp.int32)
    w = p - i0.astype(jnp.float32)
    i1 = jnp.minimum(i0 + 1, jnp.int32(scale_m1))
    return i0, i1, w


def _line_setup(c, n_m1):
    p = c * n_m1
    p = jnp.minimum(jnp.maximum(p, 0.0), n_m1)
    i0 = p.astype(jnp.int32)
    w = p - i0.astype(jnp.float32)
    i1 = jnp.minimum(i0 + 1, jnp.int32(n_m1))
    return i0, i1, w


def _body(xT, gridT, p0T, p1T, p2T, l0T, l1T, out,
          cbuf, idxbuf, wbuf, gbuf, obuf, sem):
    cid = lax.axis_index("c")
    sid = lax.axis_index("s")
    wid = sid * NC + cid
    base_w = wid * PPW
    tables = [gridT] * 8 + [p0T] * 4 + [p1T] * 4 + [p2T] * 4 + [l0T] * 2 + [l1T] * 2
    iota = lax.iota(jnp.int32, L)

    def chunk_body(t, carry):
        base = base_w + t * S
        for d in range(5):
            pltpu.sync_copy(xT.at[pl.ds(d * B + base, S)],
                            cbuf.at[pl.ds(d * S, S)])

        for g in range(NG):
            o = g * L
            c0 = cbuf[pl.ds(0 * S + o, L)]
            c1 = cbuf[pl.ds(1 * S + o, L)]
            c2 = cbuf[pl.ds(2 * S + o, L)]
            c3 = cbuf[pl.ds(3 * S + o, L)]
            c4 = cbuf[pl.ds(4 * S + o, L)]

            slot = 0
            x0, x1, wx = _axis_setup(c0, 127.0)
            y0, y1, wy = _axis_setup(c1, 127.0)
            z0, z1, wz = _axis_setup(c2, 127.0)
            xi = (x0, x1)
            wxp = (1.0 - wx, wx)
            yi = (y0, y1)
            wyp = (1.0 - wy, wy)
            zi = (z0, z1)
            wzp = (1.0 - wz, wz)
            for a in range(2):
                za = zi[a] * 128
                for b in range(2):
                    rowbase = (za + yi[b]) * 128
                    wzy = wzp[a] * wyp[b]
                    for e in range(2):
                        idxbuf[pl.ds(slot * S + o, L)] = rowbase + xi[e]
                        wbuf[pl.ds(slot * S + o, L)] = wzy * wxp[e]
                        slot += 1

            axes255 = [_axis_setup(c0, 255.0), _axis_setup(c1, 255.0),
                       _axis_setup(c2, 255.0)]
            for (pa, pb) in ((1, 2), (0, 2), (0, 1)):
                u0, u1, wu = axes255[pa]
                v0, v1, wv_ = axes255[pb]
                ui = (u0, u1)
                wup = (1.0 - wu, wu)
                vi = (v0, v1)
                wvp = (1.0 - wv_, wv_)
                for b in range(2):
                    rowbase = vi[b] * 256
                    for e in range(2):
                        idxbuf[pl.ds(slot * S + o, L)] = rowbase + ui[e]
                        wbuf[pl.ds(slot * S + o, L)] = wvp[b] * wup[e]
                        slot += 1

            for cl in (c3, c4):
                i0, i1, wl = _line_setup(cl, 63.0)
                idxbuf[pl.ds(slot * S + o, L)] = i0
                wbuf[pl.ds(slot * S + o, L)] = 1.0 - wl
                slot += 1
                idxbuf[pl.ds(slot * S + o, L)] = i1
                wbuf[pl.ds(slot * S + o, L)] = wl
                slot += 1

        descs = [pltpu.async_copy(tables[k].at[idxbuf.at[pl.ds(k * S, S)]],
                                  gbuf.at[pl.ds(k * S, S)], sem)
                 for k in range(NSLOT)]
        for dsc in descs:
            dsc.wait()

        def group_body(g, carry2):
            o = g * L
            rows = o + iota
            wv = [wbuf[pl.ds(k * S + o, L)] for k in range(NSLOT)]
            for ch in range(C):
                colc = jnp.full((L,), ch, jnp.int32)

                def gk(k):
                    return plsc.load_gather(gbuf, [k * S + rows, colc])

                acc = wv[0] * gk(0)
                for k in range(1, 8):
                    acc = acc + wv[k] * gk(k)
                for p in range(3):
                    b0 = 8 + 4 * p
                    pacc = wv[b0] * gk(b0)
                    for j in range(1, 4):
                        pacc = pacc + wv[b0 + j] * gk(b0 + j)
                    acc = acc * pacc
                l0v = wv[20] * gk(20) + wv[21] * gk(21)
                l1v = wv[22] * gk(22) + wv[23] * gk(23)
                par = l0v * l1v
                plsc.store_scatter(obuf, [rows, colc], acc)
                plsc.store_scatter(obuf, [rows, colc + C], par)
            return carry2

        lax.fori_loop(0, NG, group_body, 0)
        pltpu.sync_copy(obuf, out.at[pl.ds(base, S)])
        return carry

    lax.fori_loop(0, NCHUNK, chunk_body, 0)


_sc_call = functools.partial(
    pl.kernel,
    out_type=jax.ShapeDtypeStruct((B, OC), jnp.float32),
    mesh=plsc.VectorSubcoreMesh(core_axis_name="c", subcore_axis_name="s",
                                num_cores=NC, num_subcores=NS),
    scratch_types=[
        pltpu.VMEM((5 * S,), jnp.float32),
        pltpu.VMEM((NSLOT * S,), jnp.int32),
        pltpu.VMEM((NSLOT * S,), jnp.float32),
        pltpu.VMEM((NSLOT * S, C), jnp.float32),
        pltpu.VMEM((S, OC), jnp.float32),
        pltpu.SemaphoreType.DMA,
    ],
    compiler_params=pltpu.CompilerParams(needs_layout_passes=False,
                                         use_tc_tiling_on_sc=False),
)(_body)


def kernel(x, grid3d, plane0, plane1, plane2, line0, line1):
    xT = x.T.reshape(-1)
    gridT = jnp.transpose(grid3d, (1, 2, 3, 0)).reshape(-1, C)
    p0T = jnp.transpose(plane0, (1, 2, 0)).reshape(-1, C)
    p1T = jnp.transpose(plane1, (1, 2, 0)).reshape(-1, C)
    p2T = jnp.transpose(plane2, (1, 2, 0)).reshape(-1, C)
    l0T = line0.T
    l1T = line1.T
    return _sc_call(xT, gridT, p0T, p1T, p2T, l0T, l1T)


# speedup vs baseline: 27.4445x; 1.1826x over previous
import functools

import jax
import jax.numpy as jnp
from jax import lax
from jax.experimental import pallas as pl
from jax.experimental.pallas import tpu as pltpu
from jax.experimental.pallas import tpu_sc as plsc

B = 262144
C = 16
NC, NS, L = 2, 16, 16
NW = NC * NS
PPW = B // NW
S = 128
NCHUNK = PPW // S
NG = S // L
NST = 20
NSLOT = 24
OC = 2 * C


def _axis_setup(c, scale_m1):
    k = 0.5 * scale_m1
    p = c * k + k
    p = jnp.minimum(jnp.maximum(p, 0.0), scale_m1)
    i0 = p.astype(jnp.int32)
    w = p - i0.astype(jnp.float32)
    i1 = jnp.minimum(i0 + 1, jnp.int32(scale_m1))
    return i0, i1, w


def _line_setup(c, n_m1):
    p = c * n_m1
    p = jnp.minimum(jnp.maximum(p, 0.0), n_m1)
    i0 = p.astype(jnp.int32)
    w = p - i0.astype(jnp.float32)
    i1 = jnp.minimum(i0 + 1, jnp.int32(n_m1))
    return i0, i1, w


def _body(xT, gridT, p0T, p1T, p2T, l0T, l1T, out,
          cbuf, idxbuf, wbuf, gbuf, obuf, linebuf,
          sem_g0, sem_g1, sem_c0, sem_c1, sem_o0, sem_o1):
    cid = lax.axis_index("c")
    sid = lax.axis_index("s")
    wid = sid * NC + cid
    base_w = wid * PPW
    tables = [gridT] * 8 + [p0T] * 4 + [p1T] * 4 + [p2T] * 4
    iota = lax.iota(jnp.int32, L)
    sems_g = (sem_g0, sem_g1)
    sems_c = (sem_c0, sem_c1)
    sems_o = (sem_o0, sem_o1)

    pltpu.sync_copy(l0T, linebuf.at[pl.ds(0, 64)])
    pltpu.sync_copy(l1T, linebuf.at[pl.ds(64, 64)])

    def coords_copy(t, par):
        base = base_w + t * S
        return pltpu.make_async_copy(
            xT.at[:, pl.ds(base, S)], cbuf.at[pl.ds(par * 5, 5)], sems_c[par])

    def out_copy(t, par):
        base = base_w + t * S
        return pltpu.make_async_copy(
            obuf.at[pl.ds(par * S, S)], out.at[pl.ds(base, S)], sems_o[par])

    def gather_copy(k, par):
        return pltpu.make_async_copy(
            tables[k].at[idxbuf.at[pl.ds((par * NSLOT + k) * S, S)]],
            gbuf.at[pl.ds((par * NST + k) * S, S)], sems_g[par])

    def idxgen(par):
        co = par * 5
        io = par * NSLOT * S
        for g in range(NG):
            o = g * L
            c0 = cbuf[co + 0, pl.ds(o, L)]
            c1 = cbuf[co + 1, pl.ds(o, L)]
            c2 = cbuf[co + 2, pl.ds(o, L)]
            c3 = cbuf[co + 3, pl.ds(o, L)]
            c4 = cbuf[co + 4, pl.ds(o, L)]

            slot = 0
            x0, x1, wx = _axis_setup(c0, 127.0)
            y0, y1, wy = _axis_setup(c1, 127.0)
            z0, z1, wz = _axis_setup(c2, 127.0)
            xi = (x0, x1)
            wxp = (1.0 - wx, wx)
            yi = (y0, y1)
            wyp = (1.0 - wy, wy)
            zi = (z0, z1)
            wzp = (1.0 - wz, wz)
            for a in range(2):
                za = zi[a] * 128
                for b in range(2):
                    rowbase = (za + yi[b]) * 128
                    wzy = wzp[a] * wyp[b]
                    for e in range(2):
                        idxbuf[pl.ds(io + slot * S + o, L)] = rowbase + xi[e]
                        wbuf[pl.ds(io + slot * S + o, L)] = wzy * wxp[e]
                        slot += 1

            axes255 = [_axis_setup(c0, 255.0), _axis_setup(c1, 255.0),
                       _axis_setup(c2, 255.0)]
            for (pa, pb) in ((1, 2), (0, 2), (0, 1)):
                u0, u1, wu = axes255[pa]
                v0, v1, wv_ = axes255[pb]
                ui = (u0, u1)
                wup = (1.0 - wu, wu)
                vi = (v0, v1)
                wvp = (1.0 - wv_, wv_)
                for b in range(2):
                    rowbase = vi[b] * 256
                    for e in range(2):
                        idxbuf[pl.ds(io + slot * S + o, L)] = rowbase + ui[e]
                        wbuf[pl.ds(io + slot * S + o, L)] = wvp[b] * wup[e]
                        slot += 1

            for off, cl in ((0, c3), (64, c4)):
                i0, i1, wl = _line_setup(cl, 63.0)
                idxbuf[pl.ds(io + slot * S + o, L)] = i0 + off
                wbuf[pl.ds(io + slot * S + o, L)] = 1.0 - wl
                slot += 1
                idxbuf[pl.ds(io + slot * S + o, L)] = i1 + off
                wbuf[pl.ds(io + slot * S + o, L)] = wl
                slot += 1

    def fire_gathers(par):
        for k in range(NST):
            gather_copy(k, par).start()

    def blend(par):
        io = par * NSLOT * S
        go = par * NST * S
        oo = par * S

        def group_body(g, carry2):
            o = g * L
            rows = o + iota
            wv = [wbuf[pl.ds(io + k * S + o, L)] for k in range(NSLOT)]
            liv = [idxbuf[pl.ds(io + k * S + o, L)] for k in range(20, 24)]
            for ch in range(C):
                colc = jnp.full((L,), ch, jnp.int32)

                def gk(k):
                    return plsc.load_gather(gbuf, [go + k * S + rows, colc])

                def lk(j):
                    return plsc.load_gather(linebuf, [liv[j], colc])

                acc = wv[0] * gk(0)
                for k in range(1, 8):
                    acc = acc + wv[k] * gk(k)
                for p in range(3):
                    b0 = 8 + 4 * p
                    pacc = wv[b0] * gk(b0)
                    for j in range(1, 4):
                        pacc = pacc + wv[b0 + j] * gk(b0 + j)
                    acc = acc * pacc
                l0v = wv[20] * lk(0) + wv[21] * lk(1)
                l1v = wv[22] * lk(2) + wv[23] * lk(3)
                par_f = l0v * l1v
                plsc.store_scatter(obuf, [oo + rows, colc], acc)
                plsc.store_scatter(obuf, [oo + rows, colc + C], par_f)
            return carry2

        lax.fori_loop(0, NG, group_body, 0)

    coords_copy(0, 0).start()
    coords_copy(0, 0).wait()
    idxgen(0)
    fire_gathers(0)
    coords_copy(1, 1).start()

    def pair_body(u, carry):
        for par in range(2):
            t = u * 2 + par
            nxt = 1 - par

            @pl.when(t + 1 < NCHUNK)
            def _():
                coords_copy(t + 1, nxt).wait()
                idxgen(nxt)
                fire_gathers(nxt)

            @pl.when(t + 2 < NCHUNK)
            def _():
                coords_copy(t + 2, par).start()

            @pl.when(t >= 2)
            def _():
                out_copy(t - 2, par).wait()

            for k in range(NST):
                gather_copy(k, par).wait()
            blend(par)
            out_copy(t, par).start()
        return carry

    lax.fori_loop(0, NCHUNK // 2, pair_body, 0)
    out_copy(NCHUNK - 2, 0).wait()
    out_copy(NCHUNK - 1, 1).wait()


_sc_call = functools.partial(
    pl.kernel,
    out_type=jax.ShapeDtypeStruct((B, OC), jnp.float32),
    mesh=plsc.VectorSubcoreMesh(core_axis_name="c", subcore_axis_name="s",
                                num_cores=NC, num_subcores=NS),
    scratch_types=[
        pltpu.VMEM((2 * 5, S), jnp.float32),
        pltpu.VMEM((2 * NSLOT * S,), jnp.int32),
        pltpu.VMEM((2 * NSLOT * S,), jnp.float32),
        pltpu.VMEM((2 * NST * S, C), jnp.float32),
        pltpu.VMEM((2 * S, OC), jnp.float32),
        pltpu.VMEM((128, C), jnp.float32),
        pltpu.SemaphoreType.DMA,
        pltpu.SemaphoreType.DMA,
        pltpu.SemaphoreType.DMA,
        pltpu.SemaphoreType.DMA,
        pltpu.SemaphoreType.DMA,
        pltpu.SemaphoreType.DMA,
    ],
    compiler_params=pltpu.CompilerParams(needs_layout_passes=False,
                                         use_tc_tiling_on_sc=False),
)(_body)


def kernel(x, grid3d, plane0, plane1, plane2, line0, line1):
    xT = x.T
    gridT = jnp.transpose(grid3d, (1, 2, 3, 0)).reshape(-1, C)
    p0T = jnp.transpose(plane0, (1, 2, 0)).reshape(-1, C)
    p1T = jnp.transpose(plane1, (1, 2, 0)).reshape(-1, C)
    p2T = jnp.transpose(plane2, (1, 2, 0)).reshape(-1, C)
    l0T = line0.T
    l1T = line1.T
    return _sc_call(xT, gridT, p0T, p1T, p2T, l0T, l1T)


# speedup vs baseline: 45.1773x; 1.6461x over previous
import functools

import jax
import jax.numpy as jnp
from jax import lax
from jax.experimental import pallas as pl
from jax.experimental.pallas import tpu as pltpu
from jax.experimental.pallas import tpu_sc as plsc

B = 262144
C = 16
NC, NS, L = 2, 16, 16
NW = NC * NS
PPW = B // NW
S = 128
NCHUNK = PPW // S
NG = S // L
NST = 20
NSLOT = 24
OC = 2 * C


def _axis_setup(c, scale_m1):
    k = 0.5 * scale_m1
    p = c * k + k
    p = jnp.minimum(jnp.maximum(p, 0.0), scale_m1)
    i0 = p.astype(jnp.int32)
    w = p - i0.astype(jnp.float32)
    i1 = jnp.minimum(i0 + 1, jnp.int32(scale_m1))
    return i0, i1, w


def _line_setup(c, n_m1):
    p = c * n_m1
    p = jnp.minimum(jnp.maximum(p, 0.0), n_m1)
    i0 = p.astype(jnp.int32)
    w = p - i0.astype(jnp.float32)
    i1 = jnp.minimum(i0 + 1, jnp.int32(n_m1))
    return i0, i1, w


def _body(xT, gridT, p0T, p1T, p2T, l0T, l1T, out,
          cbuf, idxbuf, wbuf, gbuf, obuf, linebuf,
          sem_g0, sem_g1, sem_c0, sem_c1, sem_o0, sem_o1):
    cid = lax.axis_index("c")
    sid = lax.axis_index("s")
    wid = sid * NC + cid
    base_w = wid * PPW
    tables = [gridT] * 8 + [p0T] * 4 + [p1T] * 4 + [p2T] * 4
    iota = lax.iota(jnp.int32, L)
    sems_g = (sem_g0, sem_g1)
    sems_c = (sem_c0, sem_c1)
    sems_o = (sem_o0, sem_o1)

    pltpu.sync_copy(l0T, linebuf.at[pl.ds(0, 64)])
    pltpu.sync_copy(l1T, linebuf.at[pl.ds(64, 64)])

    def coords_copy(t, par):
        base = base_w + t * S
        return pltpu.make_async_copy(
            xT.at[:, pl.ds(base, S)], cbuf.at[pl.ds(par * 5, 5)], sems_c[par])

    def out_copy(t, par):
        base = base_w + t * S
        return pltpu.make_async_copy(
            obuf.at[pl.ds(par * S, S)], out.at[pl.ds(base, S)], sems_o[par])

    def gather_copy(k, par):
        return pltpu.make_async_copy(
            tables[k].at[idxbuf.at[pl.ds((par * NSLOT + k) * S, S)]],
            gbuf.at[pl.ds((par * NST + k) * S, S)], sems_g[par])

    def idxgen(par):
        co = par * 5
        io = par * NSLOT * S
        for g in range(NG):
            o = g * L
            c0 = cbuf[co + 0, pl.ds(o, L)]
            c1 = cbuf[co + 1, pl.ds(o, L)]
            c2 = cbuf[co + 2, pl.ds(o, L)]
            c3 = cbuf[co + 3, pl.ds(o, L)]
            c4 = cbuf[co + 4, pl.ds(o, L)]

            slot = 0
            x0, x1, wx = _axis_setup(c0, 127.0)
            y0, y1, wy = _axis_setup(c1, 127.0)
            z0, z1, wz = _axis_setup(c2, 127.0)
            xi = (x0, x1)
            wxp = (1.0 - wx, wx)
            yi = (y0, y1)
            wyp = (1.0 - wy, wy)
            zi = (z0, z1)
            wzp = (1.0 - wz, wz)
            for a in range(2):
                za = zi[a] * 128
                for b in range(2):
                    rowbase = (za + yi[b]) * 128
                    wzy = wzp[a] * wyp[b]
                    for e in range(2):
                        idxbuf[pl.ds(io + slot * S + o, L)] = rowbase + xi[e]
                        wbuf[pl.ds(io + slot * S + o, L)] = wzy * wxp[e]
                        slot += 1

            axes255 = [_axis_setup(c0, 255.0), _axis_setup(c1, 255.0),
                       _axis_setup(c2, 255.0)]
            for (pa, pb) in ((1, 2), (0, 2), (0, 1)):
                u0, u1, wu = axes255[pa]
                v0, v1, wv_ = axes255[pb]
                ui = (u0, u1)
                wup = (1.0 - wu, wu)
                vi = (v0, v1)
                wvp = (1.0 - wv_, wv_)
                for b in range(2):
                    rowbase = vi[b] * 256
                    for e in range(2):
                        idxbuf[pl.ds(io + slot * S + o, L)] = rowbase + ui[e]
                        wbuf[pl.ds(io + slot * S + o, L)] = wvp[b] * wup[e]
                        slot += 1

            for off, cl in ((0, c3), (64, c4)):
                i0, i1, wl = _line_setup(cl, 63.0)
                idxbuf[pl.ds(io + slot * S + o, L)] = i0 + off
                wbuf[pl.ds(io + slot * S + o, L)] = 1.0 - wl
                slot += 1
                idxbuf[pl.ds(io + slot * S + o, L)] = i1 + off
                wbuf[pl.ds(io + slot * S + o, L)] = wl
                slot += 1

    def fire_gathers(par):
        for k in range(NST):
            gather_copy(k, par).start()

    def blend(par):
        io = par * NSLOT * S
        go = par * NST * S
        oo = par * S

        def group_body(g, carry2):
            o = g * L
            rows = o + iota
            wv = [wbuf[pl.ds(io + k * S + o, L)] for k in range(NSLOT)]
            liv = [idxbuf[pl.ds(io + k * S + o, L)] for k in range(20, 24)]
            for ch in range(C):
                colv = jnp.bitwise_and(iota + ch, C - 1)

                def gk(k):
                    return plsc.load_gather(gbuf, [go + k * S + rows, colv])

                def lk(j):
                    return plsc.load_gather(linebuf, [liv[j], colv])

                acc = wv[0] * gk(0)
                for k in range(1, 8):
                    acc = acc + wv[k] * gk(k)
                for p in range(3):
                    b0 = 8 + 4 * p
                    pacc = wv[b0] * gk(b0)
                    for j in range(1, 4):
                        pacc = pacc + wv[b0 + j] * gk(b0 + j)
                    acc = acc * pacc
                l0v = wv[20] * lk(0) + wv[21] * lk(1)
                l1v = wv[22] * lk(2) + wv[23] * lk(3)
                par_f = l0v * l1v
                plsc.store_scatter(obuf, [oo + rows, colv], acc)
                plsc.store_scatter(obuf, [oo + rows, colv + C], par_f)
            return carry2

        lax.fori_loop(0, NG, group_body, 0)

    coords_copy(0, 0).start()
    coords_copy(0, 0).wait()
    idxgen(0)
    fire_gathers(0)
    coords_copy(1, 1).start()

    def pair_body(u, carry):
        for par in range(2):
            t = u * 2 + par
            nxt = 1 - par

            @pl.when(t + 1 < NCHUNK)
            def _():
                coords_copy(t + 1, nxt).wait()
                idxgen(nxt)
                fire_gathers(nxt)

            @pl.when(t + 2 < NCHUNK)
            def _():
                coords_copy(t + 2, par).start()

            @pl.when(t >= 2)
            def _():
                out_copy(t - 2, par).wait()

            for k in range(NST):
                gather_copy(k, par).wait()
            blend(par)
            out_copy(t, par).start()
        return carry

    lax.fori_loop(0, NCHUNK // 2, pair_body, 0)
    out_copy(NCHUNK - 2, 0).wait()
    out_copy(NCHUNK - 1, 1).wait()


_sc_call = functools.partial(
    pl.kernel,
    out_type=jax.ShapeDtypeStruct((B, OC), jnp.float32),
    mesh=plsc.VectorSubcoreMesh(core_axis_name="c", subcore_axis_name="s",
                                num_cores=NC, num_subcores=NS),
    scratch_types=[
        pltpu.VMEM((2 * 5, S), jnp.float32),
        pltpu.VMEM((2 * NSLOT * S,), jnp.int32),
        pltpu.VMEM((2 * NSLOT * S,), jnp.float32),
        pltpu.VMEM((2 * NST * S, C), jnp.float32),
        pltpu.VMEM((2 * S, OC), jnp.float32),
        pltpu.VMEM((128, C), jnp.float32),
        pltpu.SemaphoreType.DMA,
        pltpu.SemaphoreType.DMA,
        pltpu.SemaphoreType.DMA,
        pltpu.SemaphoreType.DMA,
        pltpu.SemaphoreType.DMA,
        pltpu.SemaphoreType.DMA,
    ],
    compiler_params=pltpu.CompilerParams(needs_layout_passes=False,
                                         use_tc_tiling_on_sc=False),
)(_body)


def kernel(x, grid3d, plane0, plane1, plane2, line0, line1):
    xT = x.T
    gridT = jnp.transpose(grid3d, (1, 2, 3, 0)).reshape(-1, C)
    p0T = jnp.transpose(plane0, (1, 2, 0)).reshape(-1, C)
    p1T = jnp.transpose(plane1, (1, 2, 0)).reshape(-1, C)
    p2T = jnp.transpose(plane2, (1, 2, 0)).reshape(-1, C)
    l0T = line0.T
    l1T = line1.T
    return _sc_call(xT, gridT, p0T, p1T, p2T, l0T, l1T)


# speedup vs baseline: 75.5604x; 1.6725x over previous
import functools

import jax
import jax.numpy as jnp
from jax import lax
from jax.experimental import pallas as pl
from jax.experimental.pallas import tpu as pltpu
from jax.experimental.pallas import tpu_sc as plsc

B = 262144
C = 16
NC, NS, L = 2, 16, 16
NW = NC * NS
PPW = B // NW
S = 128
NCHUNK = PPW // S
NG = S // L
NST = 20
NSLOT = 24
OC = 2 * C


def _axis_setup(c, scale_m1):
    k = 0.5 * scale_m1
    p = c * k + k
    p = jnp.minimum(jnp.maximum(p, 0.0), scale_m1)
    i0 = p.astype(jnp.int32)
    w = p - i0.astype(jnp.float32)
    i1 = jnp.minimum(i0 + 1, jnp.int32(scale_m1))
    return i0, i1, w


def _line_setup(c, n_m1):
    p = c * n_m1
    p = jnp.minimum(jnp.maximum(p, 0.0), n_m1)
    i0 = p.astype(jnp.int32)
    w = p - i0.astype(jnp.float32)
    i1 = jnp.minimum(i0 + 1, jnp.int32(n_m1))
    return i0, i1, w


def _body(xT, gridT, p0T, p1T, p2T, l0T, l1T, out,
          cbuf, idxbuf, wbuf, gbuf, obuf, linebuf,
          sem_g0, sem_g1, sem_c0, sem_c1, sem_o0, sem_o1):
    cid = lax.axis_index("c")
    sid = lax.axis_index("s")
    wid = sid * NC + cid
    base_w = wid * PPW
    tables = [gridT] * 8 + [p0T] * 4 + [p1T] * 4 + [p2T] * 4
    iota = lax.iota(jnp.int32, L)
    sems_g = (sem_g0, sem_g1)
    sems_c = (sem_c0, sem_c1)
    sems_o = (sem_o0, sem_o1)

    pltpu.sync_copy(l0T, linebuf.at[pl.ds(0, 64)])
    pltpu.sync_copy(l1T, linebuf.at[pl.ds(64, 64)])

    def coords_copy(t, par):
        base = base_w + t * S
        return pltpu.make_async_copy(
            xT.at[:, pl.ds(base, S)], cbuf.at[pl.ds(par * 5, 5)], sems_c[par])

    def out_copy(t, par):
        base = base_w + t * S
        return pltpu.make_async_copy(
            obuf.at[pl.ds(par * S, S)], out.at[pl.ds(base, S)], sems_o[par])

    def gather_copy(k, par):
        return pltpu.make_async_copy(
            tables[k].at[idxbuf.at[pl.ds((par * NSLOT + k) * S, S)]],
            gbuf.at[pl.ds((par * NST + k) * S, S)], sems_g[par])

    def idxgen(par):
        co = par * 5
        io = par * NSLOT * S
        for g in range(NG):
            o = g * L
            c0 = cbuf[co + 0, pl.ds(o, L)]
            c1 = cbuf[co + 1, pl.ds(o, L)]
            c2 = cbuf[co + 2, pl.ds(o, L)]
            c3 = cbuf[co + 3, pl.ds(o, L)]
            c4 = cbuf[co + 4, pl.ds(o, L)]

            slot = 0
            x0, x1, wx = _axis_setup(c0, 127.0)
            y0, y1, wy = _axis_setup(c1, 127.0)
            z0, z1, wz = _axis_setup(c2, 127.0)
            xi = (x0, x1)
            wxp = (1.0 - wx, wx)
            yi = (y0, y1)
            wyp = (1.0 - wy, wy)
            zi = (z0, z1)
            wzp = (1.0 - wz, wz)
            for a in range(2):
                za = zi[a] * 128
                for b in range(2):
                    rowbase = (za + yi[b]) * 128
                    wzy = wzp[a] * wyp[b]
                    for e in range(2):
                        idxbuf[pl.ds(io + slot * S + o, L)] = rowbase + xi[e]
                        wbuf[pl.ds(io + slot * S + o, L)] = wzy * wxp[e]
                        slot += 1

            axes255 = [_axis_setup(c0, 255.0), _axis_setup(c1, 255.0),
                       _axis_setup(c2, 255.0)]
            for (pa, pb) in ((1, 2), (0, 2), (0, 1)):
                u0, u1, wu = axes255[pa]
                v0, v1, wv_ = axes255[pb]
                ui = (u0, u1)
                wup = (1.0 - wu, wu)
                vi = (v0, v1)
                wvp = (1.0 - wv_, wv_)
                for b in range(2):
                    rowbase = vi[b] * 256
                    for e in range(2):
                        idxbuf[pl.ds(io + slot * S + o, L)] = rowbase + ui[e]
                        wbuf[pl.ds(io + slot * S + o, L)] = wvp[b] * wup[e]
                        slot += 1

            for off, cl in ((0, c3), (64, c4)):
                i0, i1, wl = _line_setup(cl, 63.0)
                idxbuf[pl.ds(io + slot * S + o, L)] = i0 + off
                wbuf[pl.ds(io + slot * S + o, L)] = 1.0 - wl
                slot += 1
                idxbuf[pl.ds(io + slot * S + o, L)] = i1 + off
                wbuf[pl.ds(io + slot * S + o, L)] = wl
                slot += 1

    def fire_gathers(par):
        for k in range(NST):
            gather_copy(k, par).start()

    def blend(par):
        io = par * NSLOT * S
        go = par * NST * S
        oo = par * S

        def group_body(g, carry2):
            o = g * L
            rows = o + iota
            wv = [wbuf[pl.ds(io + k * S + o, L)] for k in range(NSLOT)]
            liv = [idxbuf[pl.ds(io + k * S + o, L)] for k in range(20, 24)]
            for ch in range(C):
                colv = jnp.bitwise_and(iota + ch, C - 1)

                def gk(k):
                    return plsc.load_gather(gbuf, [go + k * S + rows, colv])

                def lk(j):
                    return plsc.load_gather(linebuf, [liv[j], colv])

                acc = wv[0] * gk(0)
                for k in range(1, 8):
                    acc = acc + wv[k] * gk(k)
                for p in range(3):
                    b0 = 8 + 4 * p
                    pacc = wv[b0] * gk(b0)
                    for j in range(1, 4):
                        pacc = pacc + wv[b0 + j] * gk(b0 + j)
                    acc = acc * pacc
                l0v = wv[20] * lk(0) + wv[21] * lk(1)
                l1v = wv[22] * lk(2) + wv[23] * lk(3)
                par_f = l0v * l1v
                plsc.store_scatter(obuf, [oo + rows, colv], acc)
                plsc.store_scatter(obuf, [oo + rows, colv + C], par_f)
            return carry2

        lax.fori_loop(0, NG, group_body, 0)

    coords_copy(0, 0).start()
    coords_copy(0, 0).wait()
    idxgen(0)
    fire_gathers(0)
    coords_copy(1, 1).start()

    def pair_body(u, carry):
        for par in range(2):
            t = u * 2 + par
            nxt = 1 - par

            @pl.when(t + 1 < NCHUNK)
            def _():
                coords_copy(t + 1, nxt).wait()
                idxgen(nxt)
                fire_gathers(nxt)

            @pl.when(t + 2 < NCHUNK)
            def _():
                coords_copy(t + 2, par).start()

            @pl.when(t >= 2)
            def _():
                out_copy(t - 2, par).wait()

            for k in range(NST):
                gather_copy(k, par).wait()
            blend(par)
            out_copy(t, par).start()
        return carry

    lax.fori_loop(0, NCHUNK // 2, pair_body, 0)
    out_copy(NCHUNK - 2, 0).wait()
    out_copy(NCHUNK - 1, 1).wait()


def _tr_body(grid2d, pn0, pn1, pn2, gridT, p0T, p1T, p2T,
             tbuf, tobuf, sem_i0, sem_i1, sem_o0, sem_o1):
    cid = lax.axis_index("c")
    sid = lax.axis_index("s")
    wid = sid * NC + cid
    iota = lax.iota(jnp.int32, L)
    sems_i = (sem_i0, sem_i1)
    sems_o = (sem_o0, sem_o1)
    K = 1024
    KR = K // 128

    def run_phase(src, dst, rc, n):
        def in_copies(t, par):
            r0 = (wid * n + t) * KR
            return [pltpu.make_async_copy(
                        src.at[pl.ds(c * rc + r0, KR)],
                        tbuf.at[pl.ds(par * 128 + c * KR, KR)], sems_i[par])
                    for c in range(C)]

        def out_copy(t, par):
            q = wid * n + t
            return pltpu.make_async_copy(
                tobuf.at[pl.ds(par * K, K)], dst.at[pl.ds(q * K, K)],
                sems_o[par])

        def transpose(par):
            tb = par * 128
            ob = par * K

            def gbody(g, carry):
                o = g * L
                y = lax.shift_right_logical(g, 3)
                x0 = lax.bitwise_and(g, 7) * L
                xv = x0 + iota
                posv = o + iota
                for j in range(C):
                    chv = jnp.bitwise_and(iota + j, C - 1)
                    rowv = chv * KR + (tb + y)
                    v = plsc.load_gather(tbuf, [rowv, xv])
                    plsc.store_scatter(tobuf, [ob + posv, chv], v)
                return carry

            lax.fori_loop(0, K // L, gbody, 0)

        for d in in_copies(0, 0):
            d.start()

        def pair_body(u, carry):
            for par in range(2):
                t = u * 2 + par
                nxt = 1 - par

                @pl.when(t + 1 < n)
                def _():
                    for d in in_copies(t + 1, nxt):
                        d.start()

                for d in in_copies(t, par):
                    d.wait()

                @pl.when(t >= 2)
                def _():
                    out_copy(t - 2, par).wait()

                transpose(par)
                out_copy(t, par).start()
            return carry

        lax.fori_loop(0, n // 2, pair_body, 0)
        out_copy(n - 2, 0).wait()
        out_copy(n - 1, 1).wait()

    run_phase(grid2d, gridT, 16384, 64)
    run_phase(pn0, p0T, 512, 2)
    run_phase(pn1, p1T, 512, 2)
    run_phase(pn2, p2T, 512, 2)


_tr_call = functools.partial(
    pl.kernel,
    out_type=(jax.ShapeDtypeStruct((128 * 128 * 128, C), jnp.float32),
              jax.ShapeDtypeStruct((256 * 256, C), jnp.float32),
              jax.ShapeDtypeStruct((256 * 256, C), jnp.float32),
              jax.ShapeDtypeStruct((256 * 256, C), jnp.float32)),
    mesh=plsc.VectorSubcoreMesh(core_axis_name="c", subcore_axis_name="s",
                                num_cores=NC, num_subcores=NS),
    scratch_types=[
        pltpu.VMEM((2 * 128, 128), jnp.float32),
        pltpu.VMEM((2 * 1024, C), jnp.float32),
        pltpu.SemaphoreType.DMA,
        pltpu.SemaphoreType.DMA,
        pltpu.SemaphoreType.DMA,
        pltpu.SemaphoreType.DMA,
    ],
    compiler_params=pltpu.CompilerParams(needs_layout_passes=False,
                                         use_tc_tiling_on_sc=False),
)(_tr_body)


_sc_call = functools.partial(
    pl.kernel,
    out_type=jax.ShapeDtypeStruct((B, OC), jnp.float32),
    mesh=plsc.VectorSubcoreMesh(core_axis_name="c", subcore_axis_name="s",
                                num_cores=NC, num_subcores=NS),
    scratch_types=[
        pltpu.VMEM((2 * 5, S), jnp.float32),
        pltpu.VMEM((2 * NSLOT * S,), jnp.int32),
        pltpu.VMEM((2 * NSLOT * S,), jnp.float32),
        pltpu.VMEM((2 * NST * S, C), jnp.float32),
        pltpu.VMEM((2 * S, OC), jnp.float32),
        pltpu.VMEM((128, C), jnp.float32),
        pltpu.SemaphoreType.DMA,
        pltpu.SemaphoreType.DMA,
        pltpu.SemaphoreType.DMA,
        pltpu.SemaphoreType.DMA,
        pltpu.SemaphoreType.DMA,
        pltpu.SemaphoreType.DMA,
    ],
    compiler_params=pltpu.CompilerParams(needs_layout_passes=False,
                                         use_tc_tiling_on_sc=False),
)(_body)


def kernel(x, grid3d, plane0, plane1, plane2, line0, line1):
    xT = x.T
    gridT, p0T, p1T, p2T = _tr_call(grid3d.reshape(-1, 128),
                                    plane0.reshape(-1, 128),
                                    plane1.reshape(-1, 128),
                                    plane2.reshape(-1, 128))
    l0T = line0.T
    l1T = line1.T
    return _sc_call(xT, gridT, p0T, p1T, p2T, l0T, l1T)
